# baseline (speedup 1.0000x reference)
import functools

import jax
import jax.numpy as jnp
from jax import lax
from jax.experimental import pallas as pl
from jax.experimental.pallas import tpu as pltpu

CENTER_VARIANCE = 0.1
SIZE_VARIANCE = 0.2
IMAGE_SIZE = 300.0
IOU_THRESH = 0.7
KEEP = 100
NEG = -1e30


def _prep_kernel(n_valid, cls_ref, loc_ref, pri_ref, out_ref):
    cls = cls_ref[0]
    xm = jnp.max(cls, axis=0, keepdims=True)
    unn = jnp.exp(cls - xm)
    den = jnp.sum(unn, axis=0, keepdims=True)
    soft = unn / den
    conf = jnp.max(soft, axis=0, keepdims=True)
    ciota = lax.broadcasted_iota(jnp.int32, soft.shape, 0)
    lab = jnp.min(jnp.where(soft == conf, ciota, soft.shape[0]),
                  axis=0, keepdims=True).astype(jnp.float32)

    loc = loc_ref[0]
    pr = pri_ref[...]
    l0, l1, l2, l3 = loc[0:1], loc[1:2], loc[2:3], loc[3:4]
    p0, p1, p2, p3 = pr[0:1], pr[1:2], pr[2:3], pr[3:4]
    cx = l0 * CENTER_VARIANCE * p2 + p0
    cy = l1 * CENTER_VARIANCE * p3 + p1
    w = jnp.exp(l2 * SIZE_VARIANCE) * p2
    h = jnp.exp(l3 * SIZE_VARIANCE) * p3
    x1 = (cx - w / 2.0) * IMAGE_SIZE
    y1 = (cy - h / 2.0) * IMAGE_SIZE
    x2 = (cx + w / 2.0) * IMAGE_SIZE
    y2 = (cy + h / 2.0) * IMAGE_SIZE

    lane = lax.broadcasted_iota(jnp.int32, (1, x1.shape[1]), 1)
    valid = lane < n_valid

    def vmax(a):
        return jnp.max(jnp.where(valid, a, float("-inf")))

    mc = jnp.maximum(jnp.maximum(vmax(x1), vmax(y1)),
                     jnp.maximum(vmax(x2), vmax(y2))) + 1.0
    off = lab * mc
    ox1 = x1 + off
    oy1 = y1 + off
    ox2 = x2 + off
    oy2 = y2 + off
    areas = jnp.clip(ox2 - ox1, 0.0) * jnp.clip(oy2 - oy1, 0.0)

    z = 0.0
    out_ref[0, 0:1, :] = jnp.where(valid, conf, NEG)
    out_ref[0, 1:2, :] = jnp.where(valid, ox1, z)
    out_ref[0, 2:3, :] = jnp.where(valid, oy1, z)
    out_ref[0, 3:4, :] = jnp.where(valid, ox2, z)
    out_ref[0, 4:5, :] = jnp.where(valid, oy2, z)
    out_ref[0, 5:6, :] = jnp.where(valid, areas, z)
    out_ref[0, 6:7, :] = jnp.where(valid, x1, z)
    out_ref[0, 7:8, :] = jnp.where(valid, y1, z)
    out_ref[0, 8:9, :] = jnp.where(valid, x2, z)
    out_ref[0, 9:10, :] = jnp.where(valid, y2, z)


def _nms_kernel_batched(R, B, in_ref, conf_ref, idx_ref, box_ref, sc_ref,
                        bm_ref):
    G = R // 8
    sc_ref[...] = conf_ref[...].reshape(B * R, 128)
    bm_ref[...] = jnp.max(conf_ref[...].reshape(B, G, 8, 128),
                          axis=2).reshape(B * G, 128)
    lane = lax.broadcasted_iota(jnp.int32, (1, 128), 1)
    giota3 = lax.broadcasted_iota(jnp.int32, (B, G, 128), 1)
    lrow3 = lax.broadcasted_iota(jnp.int32, (B, 8, 128), 1)
    lane3 = lax.broadcasted_iota(jnp.int32, (B, 8, 128), 2)
    BIGI = jnp.int32(2 ** 30)

    def cond(st):
        return jnp.any((st[0] < KEEP) & (st[1] > 0))

    def body(st):
        kcount, has_c, sel, kx1, ky1, kx2, ky2, kar = st
        bm3 = bm_ref[...].reshape(B, G, 128)
        mcol = jnp.max(jnp.max(bm3, axis=1), axis=1, keepdims=True)
        has = mcol > NEG
        gv = jnp.where(bm3 == mcol[:, :, None], giota3, BIGI)
        g16 = jnp.min(jnp.min(gv, axis=1), axis=1, keepdims=True)

        gbs = []
        grps = []
        for b in range(B):
            gb = jnp.sum(g16[b:b + 1, 0:1])
            gbs.append(gb)
            grps.append(sc_ref[pl.ds(b * R + gb * 8, 8), :])
        Gt = jnp.stack(grps, axis=0)

        fgs = []
        for sec in range(1, 6):
            fgs.append(jnp.stack(
                [in_ref[b, pl.ds(sec * R + gbs[b] * 8, 8), :]
                 for b in range(B)], axis=0))

        li_loc = (g16[:, :, None] * 8 + lrow3) * 128 + lane3
        geq = Gt == mcol[:, :, None]
        idx16 = jnp.min(jnp.min(jnp.where(geq, li_loc, BIGI), axis=1),
                        axis=1, keepdims=True)
        r16 = idx16 // 128
        c16 = idx16 - r16 * 128
        lm = lane == c16
        lr = r16 - g16 * 8
        sel3 = (lrow3 == lr[:, :, None]) & (lane3 == c16[:, :, None])

        def fcol(k):
            v = jnp.sum(jnp.where(sel3, fgs[k], 0.0), axis=1)
            return jnp.sum(v, axis=1, keepdims=True)

        sx1 = fcol(0)
        sy1 = fcol(1)
        sx2 = fcol(2)
        sy2 = fcol(3)
        a1 = fcol(4)

        xx1 = jnp.maximum(sx1, kx1)
        yy1 = jnp.maximum(sy1, ky1)
        xx2 = jnp.minimum(sx2, kx2)
        yy2 = jnp.minimum(sy2, ky2)
        iw = jnp.clip(xx2 - xx1, 0.0)
        ih = jnp.clip(yy2 - yy1, 0.0)
        inter = iw * ih
        iou = inter / (a1 + kar - inter + 1e-9)
        supp = jnp.any((iou > IOU_THRESH) & (lane < kcount),
                       axis=1, keepdims=True)
        keep16 = has & jnp.logical_not(supp)

        lr16 = r16 - g16 * 8
        pmask = ((lrow3 == lr16[:, :, None])
                 & (lane3 == c16[:, :, None]) & has[:, :, None])
        Gnew = jnp.where(pmask, NEG, Gt)
        for b in range(B):
            sc_ref[pl.ds(b * R + gbs[b] * 8, 8), :] = Gnew[b]
        colv = jnp.where(lane3 == c16[:, :, None], Gnew, NEG)
        colmax = jnp.max(jnp.max(colv, axis=1), axis=1, keepdims=True)
        for b in range(B):
            addr = b * G + gbs[b]
            old = bm_ref[pl.ds(addr, 1), :]
            msk = lm[b:b + 1, :] & has[b:b + 1, :]
            bm_ref[pl.ds(addr, 1), :] = jnp.where(
                msk, colmax[b:b + 1, :], old)

        am = (lane == kcount) & keep16
        sel = jnp.where(am, idx16, sel)
        kx1 = jnp.where(am, sx1, kx1)
        ky1 = jnp.where(am, sy1, ky1)
        kx2 = jnp.where(am, sx2, kx2)
        ky2 = jnp.where(am, sy2, ky2)
        kar = jnp.where(am, a1, kar)
        kinc = (keep16 & (kcount > -1)).astype(jnp.int32)
        has_i = (has | (kcount < 0)).astype(jnp.int32)
        return (kcount + kinc, has_i, sel,
                kx1, ky1, kx2, ky2, kar)

    zf = sc_ref[0:B, 0:128] * 0.0
    z32 = zf.astype(jnp.int32)
    ones_i = z32 + 1
    st = (z32, ones_i, z32, zf, zf, zf, zf, zf)
    _, _, sel, _, _, _, _, _ = lax.while_loop(cond, body, st)

    def gbody(i, bx):
        b0, b1, b2, b3 = bx
        lmi = lane == i
        idxc = jnp.sum(jnp.where(lmi, sel, 0), axis=1, keepdims=True)
        rc = idxc // 128
        cc = idxc - rc * 128
        lmc = lane == cc
        rcs = [jnp.sum(rc[b:b + 1, 0:1]) for b in range(B)]

        def fc(sec):
            rows = [in_ref[b, pl.ds(sec * R + rcs[b], 1), :]
                    for b in range(B)]
            F = jnp.concatenate(rows, axis=0)
            return jnp.sum(jnp.where(lmc, F, 0.0), axis=1, keepdims=True)

        b0 = jnp.where(lmi, fc(6), b0)
        b1 = jnp.where(lmi, fc(7), b1)
        b2 = jnp.where(lmi, fc(8), b2)
        b3 = jnp.where(lmi, fc(9), b3)
        return (b0, b1, b2, b3)

    b0, b1, b2, b3 = lax.fori_loop(0, KEEP, gbody, (zf, zf, zf, zf))
    idx_ref[...] = sel.reshape(B, 1, 128)
    box_ref[...] = jnp.stack([b0, b1, b2, b3], axis=1)


def _nms_kernel(rows, in_ref, idx_ref, box_ref, sc_ref):
    R = rows
    sc_ref[...] = in_ref[0, 0:R, :]
    iota2 = (lax.broadcasted_iota(jnp.int32, (R, 128), 0) * 128
             + lax.broadcasted_iota(jnp.int32, (R, 128), 1))
    lane = lax.broadcasted_iota(jnp.int32, (1, 128), 1)

    def pick_at(sec, r, lm):
        row = in_ref[0, pl.ds(sec * R + r, 1), :]
        return jnp.sum(jnp.where(lm, row, 0.0))

    def cond(st):
        return (st[0] < KEEP) & st[1]

    def body(st):
        kcount, alive, sel, kx1, ky1, kx2, ky2, kar, b0, b1, b2, b3 = st
        sc = sc_ref[...]
        m = jnp.max(sc)
        has = m > NEG
        idx = jnp.min(jnp.where(sc == m, iota2, jnp.int32(2 ** 30)))
        r = idx // 128
        c = idx - r * 128
        lm = lane == c

        sx1 = pick_at(1, r, lm)
        sy1 = pick_at(2, r, lm)
        sx2 = pick_at(3, r, lm)
        sy2 = pick_at(4, r, lm)
        a1 = pick_at(5, r, lm)

        xx1 = jnp.maximum(sx1, kx1)
        yy1 = jnp.maximum(sy1, ky1)
        xx2 = jnp.minimum(sx2, kx2)
        yy2 = jnp.minimum(sy2, ky2)
        iw = jnp.clip(xx2 - xx1, 0.0)
        ih = jnp.clip(yy2 - yy1, 0.0)
        inter = iw * ih
        iou = inter / (a1 + kar - inter + 1e-9)
        suppressed = jnp.any((iou > IOU_THRESH) & (lane < kcount))
        keepit = has & jnp.logical_not(suppressed)

        rowv = sc_ref[pl.ds(r, 1), :]
        sc_ref[pl.ds(r, 1), :] = jnp.where(lm & has, NEG, rowv)

        am = (lane == kcount) & keepit
        sel = jnp.where(am, idx, sel)
        kx1 = jnp.where(am, sx1, kx1)
        ky1 = jnp.where(am, sy1, ky1)
        kx2 = jnp.where(am, sx2, kx2)
        ky2 = jnp.where(am, sy2, ky2)
        kar = jnp.where(am, a1, kar)
        b0 = jnp.where(am, pick_at(6, r, lm), b0)
        b1 = jnp.where(am, pick_at(7, r, lm), b1)
        b2 = jnp.where(am, pick_at(8, r, lm), b2)
        b3 = jnp.where(am, pick_at(9, r, lm), b3)
        return (kcount + keepit.astype(jnp.int32), has, sel,
                kx1, ky1, kx2, ky2, kar, b0, b1, b2, b3)

    z32 = jnp.zeros((1, 128), jnp.int32)
    zf = jnp.zeros((1, 128), jnp.float32)
    st = (jnp.int32(0), jnp.bool_(True), z32, zf, zf, zf, zf, zf,
          zf, zf, zf, zf)
    kcount, _, sel, _, _, _, _, _, b0, b1, b2, b3 = lax.while_loop(
        cond, body, st)

    lane0 = lane == 0
    fill = lane >= kcount
    b0 = jnp.where(fill, pick_at(6, 0, lane0), b0)
    b1 = jnp.where(fill, pick_at(7, 0, lane0), b1)
    b2 = jnp.where(fill, pick_at(8, 0, lane0), b2)
    b3 = jnp.where(fill, pick_at(9, 0, lane0), b3)

    idx_ref[0, 0:1, :] = sel
    box_ref[0, 0:1, :] = b0
    box_ref[0, 1:2, :] = b1
    box_ref[0, 2:3, :] = b2
    box_ref[0, 3:4, :] = b3


def kernel(bbox_pred, cls_logits, priors):
    B, N, C = cls_logits.shape
    R = ((N + 127) // 128 + 7) // 8 * 8
    NPAD = R * 128

    clsp = jnp.pad(cls_logits, ((0, 0), (0, NPAD - N), (0, 0))).transpose(0, 2, 1)
    locp = jnp.pad(bbox_pred, ((0, 0), (0, NPAD - N), (0, 0))).transpose(0, 2, 1)
    prip = jnp.pad(priors, ((0, NPAD - N), (0, 0))).T

    prep = pl.pallas_call(
        functools.partial(_prep_kernel, N),
        grid=(B,),
        in_specs=[
            pl.BlockSpec((1, C, NPAD), lambda b: (b, 0, 0)),
            pl.BlockSpec((1, 4, NPAD), lambda b: (b, 0, 0)),
            pl.BlockSpec((4, NPAD), lambda b: (0, 0)),
        ],
        out_specs=pl.BlockSpec((1, 10, NPAD), lambda b: (b, 0, 0)),
        out_shape=jax.ShapeDtypeStruct((B, 10, NPAD), jnp.float32),
        compiler_params=pltpu.CompilerParams(
            dimension_semantics=("parallel",)),
    )(clsp, locp, prip)

    packed = prep.reshape(B, 10 * R, 128)
    conf_rows = prep[:, 0].reshape(B, R, 128)

    NSPLIT = 2
    Bh = B // NSPLIT
    idxo, boxo = pl.pallas_call(
        functools.partial(_nms_kernel_batched, R, Bh),
        grid=(NSPLIT,),
        in_specs=[
            pl.BlockSpec((Bh, 10 * R, 128), lambda i: (i, 0, 0)),
            pl.BlockSpec((Bh, R, 128), lambda i: (i, 0, 0)),
        ],
        out_specs=[
            pl.BlockSpec((Bh, 1, 128), lambda i: (i, 0, 0)),
            pl.BlockSpec((Bh, 4, 128), lambda i: (i, 0, 0)),
        ],
        out_shape=[
            jax.ShapeDtypeStruct((B, 1, 128), jnp.int32),
            jax.ShapeDtypeStruct((B, 4, 128), jnp.float32),
        ],
        scratch_shapes=[pltpu.VMEM((Bh * R, 128), jnp.float32),
                        pltpu.VMEM((Bh * (R // 8), 128), jnp.float32)],
        compiler_params=pltpu.CompilerParams(
            dimension_semantics=("parallel",)),
    )(packed, conf_rows)

    nms_indices = idxo[:, 0, :KEEP]
    nms_boxes = boxo[:, :, :KEEP].transpose(0, 2, 1)
    return nms_boxes, nms_indices


# speedup vs baseline: 1.3739x; 1.3739x over previous
import functools

import jax
import jax.numpy as jnp
from jax import lax
from jax.experimental import pallas as pl
from jax.experimental.pallas import tpu as pltpu

CENTER_VARIANCE = 0.1
SIZE_VARIANCE = 0.2
IMAGE_SIZE = 300.0
IOU_THRESH = 0.7
KEEP = 100
NEG = -1e30


def _prep_kernel(n_valid, cls_ref, loc_ref, pri_ref, out_ref):
    cls = cls_ref[0]
    xm = jnp.max(cls, axis=0, keepdims=True)
    unn = jnp.exp(cls - xm)
    den = jnp.sum(unn, axis=0, keepdims=True)
    soft = unn / den
    conf = jnp.max(soft, axis=0, keepdims=True)
    ciota = lax.broadcasted_iota(jnp.int32, soft.shape, 0)
    lab = jnp.min(jnp.where(soft == conf, ciota, soft.shape[0]),
                  axis=0, keepdims=True).astype(jnp.float32)

    loc = loc_ref[0]
    pr = pri_ref[...]
    l0, l1, l2, l3 = loc[0:1], loc[1:2], loc[2:3], loc[3:4]
    p0, p1, p2, p3 = pr[0:1], pr[1:2], pr[2:3], pr[3:4]
    cx = l0 * CENTER_VARIANCE * p2 + p0
    cy = l1 * CENTER_VARIANCE * p3 + p1
    w = jnp.exp(l2 * SIZE_VARIANCE) * p2
    h = jnp.exp(l3 * SIZE_VARIANCE) * p3
    x1 = (cx - w / 2.0) * IMAGE_SIZE
    y1 = (cy - h / 2.0) * IMAGE_SIZE
    x2 = (cx + w / 2.0) * IMAGE_SIZE
    y2 = (cy + h / 2.0) * IMAGE_SIZE

    lane = lax.broadcasted_iota(jnp.int32, (1, x1.shape[1]), 1)
    valid = lane < n_valid

    def vmax(a):
        return jnp.max(jnp.where(valid, a, float("-inf")))

    mc = jnp.maximum(jnp.maximum(vmax(x1), vmax(y1)),
                     jnp.maximum(vmax(x2), vmax(y2))) + 1.0
    off = lab * mc
    ox1 = x1 + off
    oy1 = y1 + off
    ox2 = x2 + off
    oy2 = y2 + off
    areas = jnp.clip(ox2 - ox1, 0.0) * jnp.clip(oy2 - oy1, 0.0)

    z = 0.0
    out_ref[0, 0:1, :] = jnp.where(valid, conf, NEG)
    out_ref[0, 1:2, :] = jnp.where(valid, ox1, z)
    out_ref[0, 2:3, :] = jnp.where(valid, oy1, z)
    out_ref[0, 3:4, :] = jnp.where(valid, ox2, z)
    out_ref[0, 4:5, :] = jnp.where(valid, oy2, z)
    out_ref[0, 5:6, :] = jnp.where(valid, areas, z)
    out_ref[0, 6:7, :] = jnp.where(valid, x1, z)
    out_ref[0, 7:8, :] = jnp.where(valid, y1, z)
    out_ref[0, 8:9, :] = jnp.where(valid, x2, z)
    out_ref[0, 9:10, :] = jnp.where(valid, y2, z)


def _nms_kernel_batched(R, B, in_ref, conf_ref, idx_ref, box_ref, sc_ref,
                        bm_ref):
    G = R // 8
    sc_ref[...] = conf_ref[...].reshape(B * R, 128)
    bm_ref[...] = jnp.max(conf_ref[...].reshape(B, G, 8, 128),
                          axis=2).reshape(B * G, 128)
    lane = lax.broadcasted_iota(jnp.int32, (1, 128), 1)
    giota3 = lax.broadcasted_iota(jnp.int32, (B, G, 128), 1)
    lrow3 = lax.broadcasted_iota(jnp.int32, (B, 8, 128), 1)
    lane3 = lax.broadcasted_iota(jnp.int32, (B, 8, 128), 2)
    BIGI = jnp.int32(2 ** 30)

    def cond(st):
        return jnp.any((st[0] < KEEP) & (st[1] > 0))

    def body(st):
        kcount, has_c, sel, kx1, ky1, kx2, ky2, kar = st
        bm3 = bm_ref[...].reshape(B, G, 128)
        mcol = jnp.max(jnp.max(bm3, axis=1), axis=1, keepdims=True)
        has = mcol > NEG
        gv = jnp.where(bm3 == mcol[:, :, None], giota3, BIGI)
        g16 = jnp.min(jnp.min(gv, axis=1), axis=1, keepdims=True)

        gbs = []
        grps = []
        for b in range(B):
            gb = jnp.sum(g16[b:b + 1, 0:1])
            gbs.append(gb)
            grps.append(sc_ref[pl.ds(b * R + gb * 8, 8), :])
        Gt = jnp.stack(grps, axis=0)

        li_loc = (g16[:, :, None] * 8 + lrow3) * 128 + lane3
        geq = Gt == mcol[:, :, None]
        idx16 = jnp.min(jnp.min(jnp.where(geq, li_loc, BIGI), axis=1),
                        axis=1, keepdims=True)
        r16 = idx16 // 128
        c16 = idx16 - r16 * 128
        lm = lane == c16

        rbs = []
        for b in range(B):
            rbs.append(jnp.sum(r16[b:b + 1, 0:1]))

        def fcol(sec):
            rows = [in_ref[b, pl.ds(sec * R + rbs[b], 1), :]
                    for b in range(B)]
            F = jnp.concatenate(rows, axis=0)
            return jnp.sum(jnp.where(lm, F, 0.0), axis=1, keepdims=True)

        sx1 = fcol(1)
        sy1 = fcol(2)
        sx2 = fcol(3)
        sy2 = fcol(4)
        a1 = fcol(5)

        xx1 = jnp.maximum(sx1, kx1)
        yy1 = jnp.maximum(sy1, ky1)
        xx2 = jnp.minimum(sx2, kx2)
        yy2 = jnp.minimum(sy2, ky2)
        iw = jnp.clip(xx2 - xx1, 0.0)
        ih = jnp.clip(yy2 - yy1, 0.0)
        inter = iw * ih
        iou = inter / (a1 + kar - inter + 1e-9)
        supp = jnp.any((iou > IOU_THRESH) & (lane < kcount),
                       axis=1, keepdims=True)
        keep16 = has & jnp.logical_not(supp)

        lr16 = r16 - g16 * 8
        pmask = ((lrow3 == lr16[:, :, None])
                 & (lane3 == c16[:, :, None]) & has[:, :, None])
        Gnew = jnp.where(pmask, NEG, Gt)
        for b in range(B):
            sc_ref[pl.ds(b * R + gbs[b] * 8, 8), :] = Gnew[b]
        colv = jnp.where(lane3 == c16[:, :, None], Gnew, NEG)
        colmax = jnp.max(jnp.max(colv, axis=1), axis=1, keepdims=True)
        for b in range(B):
            addr = b * G + gbs[b]
            old = bm_ref[pl.ds(addr, 1), :]
            msk = lm[b:b + 1, :] & has[b:b + 1, :]
            bm_ref[pl.ds(addr, 1), :] = jnp.where(
                msk, colmax[b:b + 1, :], old)

        am = (lane == kcount) & keep16
        sel = jnp.where(am, idx16, sel)
        kx1 = jnp.where(am, sx1, kx1)
        ky1 = jnp.where(am, sy1, ky1)
        kx2 = jnp.where(am, sx2, kx2)
        ky2 = jnp.where(am, sy2, ky2)
        kar = jnp.where(am, a1, kar)
        kinc = (keep16 & (kcount > -1)).astype(jnp.int32)
        has_i = (has | (kcount < 0)).astype(jnp.int32)
        return (kcount + kinc, has_i, sel,
                kx1, ky1, kx2, ky2, kar)

    zf = sc_ref[0:B, 0:128] * 0.0
    z32 = zf.astype(jnp.int32)
    ones_i = z32 + 1
    st = (z32, ones_i, z32, zf, zf, zf, zf, zf)
    _, _, sel, _, _, _, _, _ = lax.while_loop(cond, body, st)

    def gbody(i, bx):
        b0, b1, b2, b3 = bx
        lmi = lane == i
        idxc = jnp.sum(jnp.where(lmi, sel, 0), axis=1, keepdims=True)
        rc = idxc // 128
        cc = idxc - rc * 128
        lmc = lane == cc
        rcs = [jnp.sum(rc[b:b + 1, 0:1]) for b in range(B)]

        def fc(sec):
            rows = [in_ref[b, pl.ds(sec * R + rcs[b], 1), :]
                    for b in range(B)]
            F = jnp.concatenate(rows, axis=0)
            return jnp.sum(jnp.where(lmc, F, 0.0), axis=1, keepdims=True)

        b0 = jnp.where(lmi, fc(6), b0)
        b1 = jnp.where(lmi, fc(7), b1)
        b2 = jnp.where(lmi, fc(8), b2)
        b3 = jnp.where(lmi, fc(9), b3)
        return (b0, b1, b2, b3)

    b0, b1, b2, b3 = lax.fori_loop(0, KEEP, gbody, (zf, zf, zf, zf))
    idx_ref[...] = sel.reshape(B, 1, 128)
    box_ref[...] = jnp.stack([b0, b1, b2, b3], axis=1)


def _nms_kernel(rows, in_ref, idx_ref, box_ref, sc_ref):
    R = rows
    sc_ref[...] = in_ref[0, 0:R, :]
    iota2 = (lax.broadcasted_iota(jnp.int32, (R, 128), 0) * 128
             + lax.broadcasted_iota(jnp.int32, (R, 128), 1))
    lane = lax.broadcasted_iota(jnp.int32, (1, 128), 1)

    def pick_at(sec, r, lm):
        row = in_ref[0, pl.ds(sec * R + r, 1), :]
        return jnp.sum(jnp.where(lm, row, 0.0))

    def cond(st):
        return (st[0] < KEEP) & st[1]

    def body(st):
        kcount, alive, sel, kx1, ky1, kx2, ky2, kar, b0, b1, b2, b3 = st
        sc = sc_ref[...]
        m = jnp.max(sc)
        has = m > NEG
        idx = jnp.min(jnp.where(sc == m, iota2, jnp.int32(2 ** 30)))
        r = idx // 128
        c = idx - r * 128
        lm = lane == c

        sx1 = pick_at(1, r, lm)
        sy1 = pick_at(2, r, lm)
        sx2 = pick_at(3, r, lm)
        sy2 = pick_at(4, r, lm)
        a1 = pick_at(5, r, lm)

        xx1 = jnp.maximum(sx1, kx1)
        yy1 = jnp.maximum(sy1, ky1)
        xx2 = jnp.minimum(sx2, kx2)
        yy2 = jnp.minimum(sy2, ky2)
        iw = jnp.clip(xx2 - xx1, 0.0)
        ih = jnp.clip(yy2 - yy1, 0.0)
        inter = iw * ih
        iou = inter / (a1 + kar - inter + 1e-9)
        suppressed = jnp.any((iou > IOU_THRESH) & (lane < kcount))
        keepit = has & jnp.logical_not(suppressed)

        rowv = sc_ref[pl.ds(r, 1), :]
        sc_ref[pl.ds(r, 1), :] = jnp.where(lm & has, NEG, rowv)

        am = (lane == kcount) & keepit
        sel = jnp.where(am, idx, sel)
        kx1 = jnp.where(am, sx1, kx1)
        ky1 = jnp.where(am, sy1, ky1)
        kx2 = jnp.where(am, sx2, kx2)
        ky2 = jnp.where(am, sy2, ky2)
        kar = jnp.where(am, a1, kar)
        b0 = jnp.where(am, pick_at(6, r, lm), b0)
        b1 = jnp.where(am, pick_at(7, r, lm), b1)
        b2 = jnp.where(am, pick_at(8, r, lm), b2)
        b3 = jnp.where(am, pick_at(9, r, lm), b3)
        return (kcount + keepit.astype(jnp.int32), has, sel,
                kx1, ky1, kx2, ky2, kar, b0, b1, b2, b3)

    z32 = jnp.zeros((1, 128), jnp.int32)
    zf = jnp.zeros((1, 128), jnp.float32)
    st = (jnp.int32(0), jnp.bool_(True), z32, zf, zf, zf, zf, zf,
          zf, zf, zf, zf)
    kcount, _, sel, _, _, _, _, _, b0, b1, b2, b3 = lax.while_loop(
        cond, body, st)

    lane0 = lane == 0
    fill = lane >= kcount
    b0 = jnp.where(fill, pick_at(6, 0, lane0), b0)
    b1 = jnp.where(fill, pick_at(7, 0, lane0), b1)
    b2 = jnp.where(fill, pick_at(8, 0, lane0), b2)
    b3 = jnp.where(fill, pick_at(9, 0, lane0), b3)

    idx_ref[0, 0:1, :] = sel
    box_ref[0, 0:1, :] = b0
    box_ref[0, 1:2, :] = b1
    box_ref[0, 2:3, :] = b2
    box_ref[0, 3:4, :] = b3


def kernel(bbox_pred, cls_logits, priors):
    B, N, C = cls_logits.shape
    R = ((N + 127) // 128 + 7) // 8 * 8
    NPAD = R * 128

    clsp = jnp.pad(cls_logits, ((0, 0), (0, NPAD - N), (0, 0))).transpose(0, 2, 1)
    locp = jnp.pad(bbox_pred, ((0, 0), (0, NPAD - N), (0, 0))).transpose(0, 2, 1)
    prip = jnp.pad(priors, ((0, NPAD - N), (0, 0))).T

    prep = pl.pallas_call(
        functools.partial(_prep_kernel, N),
        grid=(B,),
        in_specs=[
            pl.BlockSpec((1, C, NPAD), lambda b: (b, 0, 0)),
            pl.BlockSpec((1, 4, NPAD), lambda b: (b, 0, 0)),
            pl.BlockSpec((4, NPAD), lambda b: (0, 0)),
        ],
        out_specs=pl.BlockSpec((1, 10, NPAD), lambda b: (b, 0, 0)),
        out_shape=jax.ShapeDtypeStruct((B, 10, NPAD), jnp.float32),
        compiler_params=pltpu.CompilerParams(
            dimension_semantics=("parallel",)),
    )(clsp, locp, prip)

    packed = prep.reshape(B, 10 * R, 128)
    conf_rows = prep[:, 0].reshape(B, R, 128)

    idxo, boxo = pl.pallas_call(
        functools.partial(_nms_kernel_batched, R, B),
        grid=(1,),
        in_specs=[
            pl.BlockSpec((B, 10 * R, 128), lambda i: (0, 0, 0)),
            pl.BlockSpec((B, R, 128), lambda i: (0, 0, 0)),
        ],
        out_specs=[
            pl.BlockSpec((B, 1, 128), lambda i: (0, 0, 0)),
            pl.BlockSpec((B, 4, 128), lambda i: (0, 0, 0)),
        ],
        out_shape=[
            jax.ShapeDtypeStruct((B, 1, 128), jnp.int32),
            jax.ShapeDtypeStruct((B, 4, 128), jnp.float32),
        ],
        scratch_shapes=[pltpu.VMEM((B * R, 128), jnp.float32),
                        pltpu.VMEM((B * (R // 8), 128), jnp.float32)],
        compiler_params=pltpu.CompilerParams(
            dimension_semantics=("arbitrary",)),
    )(packed, conf_rows)

    nms_indices = idxo[:, 0, :KEEP]
    nms_boxes = boxo[:, :, :KEEP].transpose(0, 2, 1)
    return nms_boxes, nms_indices
